# baseline (speedup 1.0000x reference)
import jax
import jax.numpy as jnp
from jax import lax
from jax.experimental import pallas as pl
from jax.experimental.pallas import tpu as pltpu
from jax.experimental.pallas import tpu_sc as plsc

N = 10000
E = 160000
D_IN = 256
D_EDGE = 16
D_OUT = 256
DH = D_OUT // 2

NS = 16
CHUNK = 80
NSLOT = 4
NCHUNK = E // CHUNK
CPT = NCHUNK // NS
NLEFT = NCHUNK - CPT * NS
NITER = CPT // NSLOT
TPT = CPT - NITER * NSLOT
RCH = CHUNK
NRFULL = N // RCH
RTAIL = N - NRFULL * RCH


def _node_body(x_ref, w_ref, r_ref, b_ref, el_ref,
               p0_ref, p1_ref, base_ref, row_ref, col_ref):
    x = x_ref[...]
    pot = jnp.dot(x, w_ref[...], preferred_element_type=jnp.float32)
    p0_ref[...] = pot[:, :DH]
    p1_ref[...] = pot[:, DH:]
    base_ref[...] = (
        jnp.dot(x, r_ref[...], preferred_element_type=jnp.float32) + b_ref[...]
    )
    el = el_ref[...]
    row_ref[...] = el[0, :].reshape(1, 1, -1)
    col_ref[...] = el[1, :].reshape(1, 1, -1)


def _node_transform(node_mat, node_weight, root, bias2d, e_list):
    bm = 2000
    nb = N // bm
    be = E // nb
    grid = (nb,)
    return pl.pallas_call(
        _node_body,
        grid=grid,
        in_specs=[
            pl.BlockSpec((bm, D_IN), lambda i: (i, 0)),
            pl.BlockSpec((D_IN, D_OUT), lambda i: (0, 0)),
            pl.BlockSpec((D_IN, D_OUT), lambda i: (0, 0)),
            pl.BlockSpec((1, D_OUT), lambda i: (0, 0)),
            pl.BlockSpec((2, be), lambda i: (0, i)),
        ],
        out_specs=[
            pl.BlockSpec((bm, DH), lambda i: (i, 0)),
            pl.BlockSpec((bm, DH), lambda i: (i, 0)),
            pl.BlockSpec((bm, D_OUT), lambda i: (i, 0)),
            pl.BlockSpec((1, 1, be), lambda i: (i, 0, 0)),
            pl.BlockSpec((1, 1, be), lambda i: (i, 0, 0)),
        ],
        out_shape=[
            jax.ShapeDtypeStruct((N, DH), jnp.float32),
            jax.ShapeDtypeStruct((N, DH), jnp.float32),
            jax.ShapeDtypeStruct((N, D_OUT), jnp.float32),
            jax.ShapeDtypeStruct((nb, 1, be), jnp.int32),
            jax.ShapeDtypeStruct((nb, 1, be), jnp.int32),
        ],
    )(node_mat, node_weight, root, bias2d, e_list)


def _edge_body(et_ref, w_ref, o_ref):
    o_ref[...] = jnp.maximum(
        lax.dot_general(
            et_ref[...], w_ref[...], (((0,), (0,)), ((), ())),
            preferred_element_type=jnp.float32,
        ),
        0.0,
    )


def _edge_mlp(edge_mat_t, edge_lay_1):
    bm = 3200
    grid = (E // bm,)
    return pl.pallas_call(
        _edge_body,
        grid=grid,
        in_specs=[
            pl.BlockSpec((D_EDGE, bm), lambda i: (0, i)),
            pl.BlockSpec((D_EDGE, D_OUT), lambda i: (0, 0)),
        ],
        out_specs=pl.BlockSpec((bm, D_OUT), lambda i: (i, 0)),
        out_shape=jax.ShapeDtypeStruct((E, D_OUT), jnp.float32),
    )(edge_mat_t, edge_lay_1)


NRITER = (NRFULL + NS - 1) // NS
NRFR = NRFULL // NS
assert NRITER >= 2 and NRITER - 1 >= NRFR >= NRITER - 2


def _accum_init(c, s, src, accum, stages, sl, st):
    def load(i, b):
        pltpu.async_copy(
            src.at[pl.ds((i * NS + s) * RCH, RCH), pl.ds(c * DH, DH)],
            stages[b], sl[b])

    def wait_load(i, b):
        pltpu.make_async_copy(
            src.at[pl.ds((i * NS + s) * RCH, RCH), pl.ds(c * DH, DH)],
            stages[b], sl[b]).wait()

    def store(i, b):
        pltpu.async_copy(
            stages[b], accum.at[pl.ds((i * NS + s) * RCH, RCH)], st[b])

    def wait_store(i, b):
        pltpu.make_async_copy(
            stages[b], accum.at[pl.ds((i * NS + s) * RCH, RCH)], st[b]).wait()

    for i in range(NRITER):
        b = i % 2
        if i >= 2:
            wait_store(i - 2, b)
        if i < NRFR:
            load(i, b)
        else:
            @pl.when(i * NS + s < NRFULL)
            def _(i=i, b=b):
                load(i, b)
        if i >= 1:
            wait_load(i - 1, 1 - b)
            store(i - 1, 1 - b)
    last, bl = NRITER - 1, (NRITER - 1) % 2
    if last < NRFR:
        wait_load(last, bl)
        store(last, bl)
        wait_store(last, bl)
    else:
        @pl.when(last * NS + s < NRFULL)
        def _():
            wait_load(last, bl)
            store(last, bl)
            wait_store(last, bl)
    wait_store(last - 1, 1 - bl)


def _accum_writeout(c, s, accum, dst, stages, sl, st):
    def load(i, b):
        pltpu.async_copy(
            accum.at[pl.ds((i * NS + s) * RCH, RCH)], stages[b], sl[b])

    def wait_load(i, b):
        pltpu.make_async_copy(
            accum.at[pl.ds((i * NS + s) * RCH, RCH)], stages[b], sl[b]).wait()

    def store(i, b):
        pltpu.async_copy(
            stages[b], dst.at[pl.ds((i * NS + s) * RCH, RCH), pl.ds(c * DH, DH)],
            st[b])

    def wait_store(i, b):
        pltpu.make_async_copy(
            stages[b], dst.at[pl.ds((i * NS + s) * RCH, RCH), pl.ds(c * DH, DH)],
            st[b]).wait()

    for i in range(NRITER):
        b = i % 2
        if i >= 2:
            wait_store(i - 2, b)
        if i < NRFR:
            load(i, b)
        else:
            @pl.when(i * NS + s < NRFULL)
            def _(i=i, b=b):
                load(i, b)
        if i >= 1:
            wait_load(i - 1, 1 - b)
            store(i - 1, 1 - b)
    last, bl = NRITER - 1, (NRITER - 1) % 2
    if last < NRFR:
        wait_load(last, bl)
        store(last, bl)
        wait_store(last, bl)
    else:
        @pl.when(last * NS + s < NRFULL)
        def _():
            wait_load(last, bl)
            store(last, bl)
            wait_store(last, bl)
    wait_store(last - 1, 1 - bl)


def _sc_pot_body(pot0, pot1, base, row_hbm, col_hbm, out,
                 ib, gbuf, accum,
                 si0, si1, si2, si3, sg0, sg1, sg2, sg3,
                 ss0, ss1, ss2, ss3):
    c = lax.axis_index("c")
    s = lax.axis_index("s")
    si = (si0, si1, si2, si3)
    sg = (sg0, sg1, sg2, sg3)
    ss = (ss0, ss1, ss2, ss3)

    _accum_init(c, s, base, accum, (gbuf.at[0], gbuf.at[1]),
                (si[0], si[1]), (ss[0], ss[1]))
    plsc.subcore_barrier()

    def edge_it(t, carry):
        e0s = [((t * NSLOT + u) * NS + s) * CHUNK for u in range(NSLOT)]
        for u in range(NSLOT):
            @pl.when(t > 0)
            def _(u=u):
                pltpu.make_async_copy(
                    gbuf.at[u], accum.at[ib.at[u, 0]], ss[u]).wait()

            pltpu.async_copy(row_hbm.at[pl.ds(e0s[u], CHUNK)], ib.at[u, 0], si[u])
            pltpu.async_copy(col_hbm.at[pl.ds(e0s[u], CHUNK)], ib.at[u, 1], si[u])
        for u in range(NSLOT):
            pltpu.make_async_copy(
                row_hbm.at[pl.ds(e0s[u], CHUNK)], ib.at[u, 0], si[u]).wait()
            pltpu.make_async_copy(
                col_hbm.at[pl.ds(e0s[u], CHUNK)], ib.at[u, 1], si[u]).wait()

            @pl.when(c == 0)
            def _(u=u):
                pltpu.async_copy(pot0.at[ib.at[u, 1]], gbuf.at[u], sg[u])

            @pl.when(c == 1)
            def _(u=u):
                pltpu.async_copy(pot1.at[ib.at[u, 1]], gbuf.at[u], sg[u])

        for u in range(NSLOT):
            pltpu.make_async_copy(pot0.at[ib.at[u, 1]], gbuf.at[u], sg[u]).wait()
            pltpu.async_copy(gbuf.at[u], accum.at[ib.at[u, 0]], ss[u], add=True)
        return carry

    lax.fori_loop(0, NITER, edge_it, 0)
    for u in range(NSLOT):
        pltpu.make_async_copy(gbuf.at[u], accum.at[ib.at[u, 0]], ss[u]).wait()

    def _pot_one(e0):
        pltpu.sync_copy(row_hbm.at[pl.ds(e0, CHUNK)], ib.at[0, 0])
        pltpu.sync_copy(col_hbm.at[pl.ds(e0, CHUNK)], ib.at[0, 1])

        @pl.when(c == 0)
        def _():
            pltpu.async_copy(pot0.at[ib.at[0, 1]], gbuf.at[0], si[0]).wait()

        @pl.when(c == 1)
        def _():
            pltpu.async_copy(pot1.at[ib.at[0, 1]], gbuf.at[0], si[0]).wait()

        pltpu.sync_copy(gbuf.at[0], accum.at[ib.at[0, 0]], add=True)

    for w in range(TPT):
        _pot_one(((NITER * NSLOT + w) * NS + s) * CHUNK)
    if NLEFT:
        @pl.when(s < NLEFT)
        def _():
            _pot_one((CPT * NS + s) * CHUNK)

    plsc.subcore_barrier()
    _accum_writeout(c, s, accum, out, (gbuf.at[0], gbuf.at[1]),
                    (si[0], si[1]), (ss[0], ss[1]))


def _sc_emlp_body(partial, emlp, col_hbm, out,
                  idxc, ebuf, accum,
                  si0, si1, si2, si3, se0, se1, se2, se3,
                  ss0, ss1, ss2, ss3):
    c = lax.axis_index("c")
    s = lax.axis_index("s")
    si = (si0, si1, si2, si3)
    se = (se0, se1, se2, se3)
    ss = (ss0, ss1, ss2, ss3)

    _accum_init(c, s, partial, accum, (ebuf.at[0], ebuf.at[1]),
                (si[0], si[1]), (ss[0], ss[1]))
    plsc.subcore_barrier()

    def edge_it(t, carry):
        e0s = [((t * NSLOT + u) * NS + s) * CHUNK for u in range(NSLOT)]
        for u in range(NSLOT):
            @pl.when(t > 0)
            def _(u=u):
                pltpu.make_async_copy(
                    ebuf.at[u], accum.at[idxc.at[u]], ss[u]).wait()

            pltpu.async_copy(col_hbm.at[pl.ds(e0s[u], CHUNK)], idxc.at[u], si[u])
            pltpu.async_copy(
                emlp.at[pl.ds(e0s[u], CHUNK), pl.ds(c * DH, DH)], ebuf.at[u],
                se[u])
        for u in range(NSLOT):
            pltpu.make_async_copy(
                col_hbm.at[pl.ds(e0s[u], CHUNK)], idxc.at[u], si[u]).wait()
            pltpu.make_async_copy(
                emlp.at[pl.ds(e0s[u], CHUNK), pl.ds(c * DH, DH)], ebuf.at[u],
                se[u]).wait()
            pltpu.async_copy(ebuf.at[u], accum.at[idxc.at[u]], ss[u], add=True)
        return carry

    lax.fori_loop(0, NITER, edge_it, 0)
    for u in range(NSLOT):
        pltpu.make_async_copy(ebuf.at[u], accum.at[idxc.at[u]], ss[u]).wait()

    def _emlp_one(e0):
        pltpu.sync_copy(col_hbm.at[pl.ds(e0, CHUNK)], idxc.at[0])
        pltpu.sync_copy(emlp.at[pl.ds(e0, CHUNK), pl.ds(c * DH, DH)], ebuf.at[0])
        pltpu.sync_copy(ebuf.at[0], accum.at[idxc.at[0]], add=True)

    for w in range(TPT):
        _emlp_one(((NITER * NSLOT + w) * NS + s) * CHUNK)
    if NLEFT:
        @pl.when(s < NLEFT)
        def _():
            _emlp_one((CPT * NS + s) * CHUNK)

    plsc.subcore_barrier()
    _accum_writeout(c, s, accum, out, (ebuf.at[0], ebuf.at[1]),
                    (si[0], si[1]), (ss[0], ss[1]))


def _sc_pot_scatter(pot0, pot1, base, row, col):
    mesh = plsc.VectorSubcoreMesh(core_axis_name="c", subcore_axis_name="s")
    k = pl.kernel(
        _sc_pot_body,
        mesh=mesh,
        out_type=jax.ShapeDtypeStruct((N, D_OUT), jnp.float32),
        scratch_types=[
            pltpu.VMEM((NSLOT, 2, CHUNK), jnp.int32),
            pltpu.VMEM((NSLOT, CHUNK, DH), jnp.float32),
            pltpu.VMEM_SHARED((N, DH), jnp.float32),
        ] + [pltpu.SemaphoreType.DMA] * 12,
    )
    return k(pot0, pot1, base, row, col)


def _sc_emlp_scatter(partial, emlp, col):
    mesh = plsc.VectorSubcoreMesh(core_axis_name="c", subcore_axis_name="s")
    k = pl.kernel(
        _sc_emlp_body,
        mesh=mesh,
        out_type=jax.ShapeDtypeStruct((N, D_OUT), jnp.float32),
        scratch_types=[
            pltpu.VMEM((NSLOT, CHUNK), jnp.int32),
            pltpu.VMEM((NSLOT, CHUNK, DH), jnp.float32),
            pltpu.VMEM_SHARED((N, DH), jnp.float32),
        ] + [pltpu.SemaphoreType.DMA] * 12,
    )
    return k(partial, emlp, col)


@jax.jit
def kernel(node_mat, edge_mat, e_list, node_weight, edge_lay_1, root, bias):
    pot0, pot1, base, row3, col3 = _node_transform(
        node_mat, node_weight, root, bias.reshape(1, D_OUT), e_list
    )
    emlp = _edge_mlp(edge_mat.T, edge_lay_1)
    row = row3.reshape(E)
    col = col3.reshape(E)
    partial = _sc_pot_scatter(pot0, pot1, base, row, col)
    return _sc_emlp_scatter(partial, emlp, col)


# speedup vs baseline: 1.0021x; 1.0021x over previous
import jax
import jax.numpy as jnp
from jax import lax
from jax.experimental import pallas as pl
from jax.experimental.pallas import tpu as pltpu
from jax.experimental.pallas import tpu_sc as plsc

N = 10000
E = 160000
D_IN = 256
D_EDGE = 16
D_OUT = 256
DH = D_OUT // 2

NS = 16
CHUNK = 80
NSLOT = 4
NCHUNK = E // CHUNK
CPT = NCHUNK // NS
NLEFT = NCHUNK - CPT * NS
NITER = CPT // NSLOT
TPT = CPT - NITER * NSLOT
RCH = CHUNK
NRFULL = N // RCH
RTAIL = N - NRFULL * RCH


def _node_body(x_ref, w_ref, r_ref, b_ref, p0_ref, p1_ref, base_ref):
    x = x_ref[...]
    pot = jnp.dot(x, w_ref[...], preferred_element_type=jnp.float32)
    p0_ref[...] = pot[:, :DH]
    p1_ref[...] = pot[:, DH:]
    base_ref[...] = (
        jnp.dot(x, r_ref[...], preferred_element_type=jnp.float32) + b_ref[...]
    )


def _node_transform(node_mat, node_weight, root, bias2d):
    bm = 2000
    grid = (N // bm,)
    return pl.pallas_call(
        _node_body,
        grid=grid,
        in_specs=[
            pl.BlockSpec((bm, D_IN), lambda i: (i, 0)),
            pl.BlockSpec((D_IN, D_OUT), lambda i: (0, 0)),
            pl.BlockSpec((D_IN, D_OUT), lambda i: (0, 0)),
            pl.BlockSpec((1, D_OUT), lambda i: (0, 0)),
        ],
        out_specs=[
            pl.BlockSpec((bm, DH), lambda i: (i, 0)),
            pl.BlockSpec((bm, DH), lambda i: (i, 0)),
            pl.BlockSpec((bm, D_OUT), lambda i: (i, 0)),
        ],
        out_shape=[
            jax.ShapeDtypeStruct((N, DH), jnp.float32),
            jax.ShapeDtypeStruct((N, DH), jnp.float32),
            jax.ShapeDtypeStruct((N, D_OUT), jnp.float32),
        ],
    )(node_mat, node_weight, root, bias2d)


def _edge_body(et_ref, w_ref, o_ref):
    o_ref[...] = jnp.maximum(
        lax.dot_general(
            et_ref[...], w_ref[...], (((0,), (0,)), ((), ())),
            preferred_element_type=jnp.float32,
        ),
        0.0,
    )


def _edge_mlp(edge_mat_t, edge_lay_1):
    bm = 3200
    grid = (E // bm,)
    return pl.pallas_call(
        _edge_body,
        grid=grid,
        in_specs=[
            pl.BlockSpec((D_EDGE, bm), lambda i: (0, i)),
            pl.BlockSpec((D_EDGE, D_OUT), lambda i: (0, 0)),
        ],
        out_specs=pl.BlockSpec((bm, D_OUT), lambda i: (i, 0)),
        out_shape=jax.ShapeDtypeStruct((E, D_OUT), jnp.float32),
    )(edge_mat_t, edge_lay_1)


NRITER = (NRFULL + NS - 1) // NS
NRFR = NRFULL // NS
assert NRITER >= 2 and NRITER - 1 >= NRFR >= NRITER - 2


def _accum_init(c, s, src, accum, stages, sl, st):
    def load(i, b):
        pltpu.async_copy(
            src.at[pl.ds((i * NS + s) * RCH, RCH), pl.ds(c * DH, DH)],
            stages[b], sl[b])

    def wait_load(i, b):
        pltpu.make_async_copy(
            src.at[pl.ds((i * NS + s) * RCH, RCH), pl.ds(c * DH, DH)],
            stages[b], sl[b]).wait()

    def store(i, b):
        pltpu.async_copy(
            stages[b], accum.at[pl.ds((i * NS + s) * RCH, RCH)], st[b])

    def wait_store(i, b):
        pltpu.make_async_copy(
            stages[b], accum.at[pl.ds((i * NS + s) * RCH, RCH)], st[b]).wait()

    for i in range(NRITER):
        b = i % 2
        if i >= 2:
            wait_store(i - 2, b)
        if i < NRFR:
            load(i, b)
        else:
            @pl.when(i * NS + s < NRFULL)
            def _(i=i, b=b):
                load(i, b)
        if i >= 1:
            wait_load(i - 1, 1 - b)
            store(i - 1, 1 - b)
    last, bl = NRITER - 1, (NRITER - 1) % 2
    if last < NRFR:
        wait_load(last, bl)
        store(last, bl)
        wait_store(last, bl)
    else:
        @pl.when(last * NS + s < NRFULL)
        def _():
            wait_load(last, bl)
            store(last, bl)
            wait_store(last, bl)
    wait_store(last - 1, 1 - bl)


def _accum_writeout(c, s, accum, dst, stages, sl, st):
    def load(i, b):
        pltpu.async_copy(
            accum.at[pl.ds((i * NS + s) * RCH, RCH)], stages[b], sl[b])

    def wait_load(i, b):
        pltpu.make_async_copy(
            accum.at[pl.ds((i * NS + s) * RCH, RCH)], stages[b], sl[b]).wait()

    def store(i, b):
        pltpu.async_copy(
            stages[b], dst.at[pl.ds((i * NS + s) * RCH, RCH), pl.ds(c * DH, DH)],
            st[b])

    def wait_store(i, b):
        pltpu.make_async_copy(
            stages[b], dst.at[pl.ds((i * NS + s) * RCH, RCH), pl.ds(c * DH, DH)],
            st[b]).wait()

    for i in range(NRITER):
        b = i % 2
        if i >= 2:
            wait_store(i - 2, b)
        if i < NRFR:
            load(i, b)
        else:
            @pl.when(i * NS + s < NRFULL)
            def _(i=i, b=b):
                load(i, b)
        if i >= 1:
            wait_load(i - 1, 1 - b)
            store(i - 1, 1 - b)
    last, bl = NRITER - 1, (NRITER - 1) % 2
    if last < NRFR:
        wait_load(last, bl)
        store(last, bl)
        wait_store(last, bl)
    else:
        @pl.when(last * NS + s < NRFULL)
        def _():
            wait_load(last, bl)
            store(last, bl)
            wait_store(last, bl)
    wait_store(last - 1, 1 - bl)


def _sc_pot_body(pot0, pot1, base, row_hbm, col_hbm, out,
                 ib, gbuf, accum,
                 si0, si1, si2, si3, sg0, sg1, sg2, sg3,
                 ss0, ss1, ss2, ss3):
    c = lax.axis_index("c")
    s = lax.axis_index("s")
    si = (si0, si1, si2, si3)
    sg = (sg0, sg1, sg2, sg3)
    ss = (ss0, ss1, ss2, ss3)

    _accum_init(c, s, base, accum, (gbuf.at[0], gbuf.at[1]),
                (si[0], si[1]), (ss[0], ss[1]))
    plsc.subcore_barrier()

    def edge_it(t, carry):
        e0s = [((t * NSLOT + u) * NS + s) * CHUNK for u in range(NSLOT)]
        for u in range(NSLOT):
            @pl.when(t > 0)
            def _(u=u):
                pltpu.make_async_copy(
                    gbuf.at[u], accum.at[ib.at[u, 0]], ss[u]).wait()

            pltpu.async_copy(row_hbm.at[pl.ds(e0s[u], CHUNK)], ib.at[u, 0], si[u])
            pltpu.async_copy(col_hbm.at[pl.ds(e0s[u], CHUNK)], ib.at[u, 1], si[u])
        for u in range(NSLOT):
            pltpu.make_async_copy(
                row_hbm.at[pl.ds(e0s[u], CHUNK)], ib.at[u, 0], si[u]).wait()
            pltpu.make_async_copy(
                col_hbm.at[pl.ds(e0s[u], CHUNK)], ib.at[u, 1], si[u]).wait()

            @pl.when(c == 0)
            def _(u=u):
                pltpu.async_copy(pot0.at[ib.at[u, 1]], gbuf.at[u], sg[u])

            @pl.when(c == 1)
            def _(u=u):
                pltpu.async_copy(pot1.at[ib.at[u, 1]], gbuf.at[u], sg[u])

        for u in range(NSLOT):
            pltpu.make_async_copy(pot0.at[ib.at[u, 1]], gbuf.at[u], sg[u]).wait()
            pltpu.async_copy(gbuf.at[u], accum.at[ib.at[u, 0]], ss[u], add=True)
        return carry

    lax.fori_loop(0, NITER, edge_it, 0)
    for u in range(NSLOT):
        pltpu.make_async_copy(gbuf.at[u], accum.at[ib.at[u, 0]], ss[u]).wait()

    def _pot_one(e0):
        pltpu.sync_copy(row_hbm.at[pl.ds(e0, CHUNK)], ib.at[0, 0])
        pltpu.sync_copy(col_hbm.at[pl.ds(e0, CHUNK)], ib.at[0, 1])

        @pl.when(c == 0)
        def _():
            pltpu.async_copy(pot0.at[ib.at[0, 1]], gbuf.at[0], si[0]).wait()

        @pl.when(c == 1)
        def _():
            pltpu.async_copy(pot1.at[ib.at[0, 1]], gbuf.at[0], si[0]).wait()

        pltpu.sync_copy(gbuf.at[0], accum.at[ib.at[0, 0]], add=True)

    for w in range(TPT):
        _pot_one(((NITER * NSLOT + w) * NS + s) * CHUNK)
    if NLEFT:
        @pl.when(s < NLEFT)
        def _():
            _pot_one((CPT * NS + s) * CHUNK)

    plsc.subcore_barrier()
    _accum_writeout(c, s, accum, out, (gbuf.at[0], gbuf.at[1]),
                    (si[0], si[1]), (ss[0], ss[1]))


def _sc_emlp_body(partial, emlp, col_hbm, out,
                  idxc, ebuf, accum,
                  si0, si1, si2, si3, se0, se1, se2, se3,
                  ss0, ss1, ss2, ss3):
    c = lax.axis_index("c")
    s = lax.axis_index("s")
    si = (si0, si1, si2, si3)
    se = (se0, se1, se2, se3)
    ss = (ss0, ss1, ss2, ss3)

    _accum_init(c, s, partial, accum, (ebuf.at[0], ebuf.at[1]),
                (si[0], si[1]), (ss[0], ss[1]))
    plsc.subcore_barrier()

    def edge_it(t, carry):
        e0s = [((t * NSLOT + u) * NS + s) * CHUNK for u in range(NSLOT)]
        for u in range(NSLOT):
            @pl.when(t > 0)
            def _(u=u):
                pltpu.make_async_copy(
                    ebuf.at[u], accum.at[idxc.at[u]], ss[u]).wait()

            pltpu.async_copy(col_hbm.at[pl.ds(e0s[u], CHUNK)], idxc.at[u], si[u])
            pltpu.async_copy(
                emlp.at[pl.ds(e0s[u], CHUNK), pl.ds(c * DH, DH)], ebuf.at[u],
                se[u])
        for u in range(NSLOT):
            pltpu.make_async_copy(
                col_hbm.at[pl.ds(e0s[u], CHUNK)], idxc.at[u], si[u]).wait()
            pltpu.make_async_copy(
                emlp.at[pl.ds(e0s[u], CHUNK), pl.ds(c * DH, DH)], ebuf.at[u],
                se[u]).wait()
            pltpu.async_copy(ebuf.at[u], accum.at[idxc.at[u]], ss[u], add=True)
        return carry

    lax.fori_loop(0, NITER, edge_it, 0)
    for u in range(NSLOT):
        pltpu.make_async_copy(ebuf.at[u], accum.at[idxc.at[u]], ss[u]).wait()

    def _emlp_one(e0):
        pltpu.sync_copy(col_hbm.at[pl.ds(e0, CHUNK)], idxc.at[0])
        pltpu.sync_copy(emlp.at[pl.ds(e0, CHUNK), pl.ds(c * DH, DH)], ebuf.at[0])
        pltpu.sync_copy(ebuf.at[0], accum.at[idxc.at[0]], add=True)

    for w in range(TPT):
        _emlp_one(((NITER * NSLOT + w) * NS + s) * CHUNK)
    if NLEFT:
        @pl.when(s < NLEFT)
        def _():
            _emlp_one((CPT * NS + s) * CHUNK)

    plsc.subcore_barrier()
    _accum_writeout(c, s, accum, out, (ebuf.at[0], ebuf.at[1]),
                    (si[0], si[1]), (ss[0], ss[1]))


def _sc_pot_scatter(pot0, pot1, base, row, col):
    mesh = plsc.VectorSubcoreMesh(core_axis_name="c", subcore_axis_name="s")
    k = pl.kernel(
        _sc_pot_body,
        mesh=mesh,
        out_type=jax.ShapeDtypeStruct((N, D_OUT), jnp.float32),
        scratch_types=[
            pltpu.VMEM((NSLOT, 2, CHUNK), jnp.int32),
            pltpu.VMEM((NSLOT, CHUNK, DH), jnp.float32),
            pltpu.VMEM_SHARED((N, DH), jnp.float32),
        ] + [pltpu.SemaphoreType.DMA] * 12,
    )
    return k(pot0, pot1, base, row, col)


def _sc_emlp_scatter(partial, emlp, col):
    mesh = plsc.VectorSubcoreMesh(core_axis_name="c", subcore_axis_name="s")
    k = pl.kernel(
        _sc_emlp_body,
        mesh=mesh,
        out_type=jax.ShapeDtypeStruct((N, D_OUT), jnp.float32),
        scratch_types=[
            pltpu.VMEM((NSLOT, CHUNK), jnp.int32),
            pltpu.VMEM((NSLOT, CHUNK, DH), jnp.float32),
            pltpu.VMEM_SHARED((N, DH), jnp.float32),
        ] + [pltpu.SemaphoreType.DMA] * 12,
    )
    return k(partial, emlp, col)


@jax.jit
def kernel(node_mat, edge_mat, e_list, node_weight, edge_lay_1, root, bias):
    pot0, pot1, base = _node_transform(
        node_mat, node_weight, root, bias.reshape(1, D_OUT)
    )
    emlp = _edge_mlp(edge_mat.T, edge_lay_1)
    row = e_list[0]
    col = e_list[1]
    partial = _sc_pot_scatter(pot0, pot1, base, row, col)
    return _sc_emlp_scatter(partial, emlp, col)
